# Initial kernel scaffold; baseline (speedup 1.0000x reference)
import jax
import jax.numpy as jnp
from jax.experimental import pallas as pl


def kernel(observations, actions, rewards, next_observations, observations_in, actions_in, rewards_in, next_observations_in, indices, batch_size):
    raise NotImplementedError("write your pallas kernel here")


# speedup vs baseline: 1.2734x; 1.2734x over previous
import jax
import jax.numpy as jnp
from jax import lax
from jax.experimental import pallas as pl
from jax.experimental.pallas import tpu as pltpu
from jax.experimental.pallas import tpu_sc as plsc

N_ENVS = 1024
BUF = 1024
D_OBS = 64
D_ACT = 16
BATCH = 256
D_OUT = D_OBS + D_ACT + 1 + D_OBS
L = 16
N_WORKERS = 32
ENV_PER_W = N_ENVS // N_WORKERS
HALF = BATCH // 2
ENV_OUT = BATCH * D_OUT


def _body(obs_p, act_f, rew2, nobs_p, obs_in, act_in, rew_in1, nobs_in,
          idx_hbm, out_hbm,
          idxv, pairv, obs_g, nobs_g, actv, outv, rewrow, rinv,
          oin, ain, nin, sem):
    wid = lax.axis_index("s") * 2 + lax.axis_index("c")
    iota16 = lax.iota(jnp.int32, L)
    pltpu.sync_copy(rew_in1, rinv)

    def per_env(i, _):
        e = wid * ENV_PER_W + i
        pltpu.sync_copy(idx_hbm.at[e], idxv)
        pltpu.sync_copy(obs_in.at[e], oin)
        pltpu.sync_copy(act_in.at[e], ain)
        pltpu.sync_copy(nobs_in.at[e], nin)
        pltpu.sync_copy(rew2.at[e], rewrow)
        cp_act = pltpu.async_copy(act_f.at[e], actv, sem)

        ebase = (e * (BUF // 2)).astype(jnp.int32)
        nz = jnp.zeros((L,), jnp.int32)
        for c in range(BATCH // L):
            v = idxv[pl.ds(c * L, L)]
            h, r = divmod(c, HALF // L)
            pairv[h, pl.ds(r * L, L)] = ebase + lax.shift_right_logical(v, 1)
            nz = nz + jnp.where(v == 0, 1, 0).astype(jnp.int32)
        any0 = jnp.sum(nz) > 0

        rbc = plsc.load_gather(rinv, [jnp.full((L,), 1, jnp.int32) * e])
        for c in range(BATCH // L):
            ic = idxv[pl.ds(c * L, L)]
            g = plsc.load_gather(rewrow, [ic])
            g = jnp.where(ic == 0, rbc, g)
            plsc.store_scatter(
                outv, [(iota16 + (c * L)) * D_OUT + (D_OBS + D_ACT)], g)

        cp_act.wait()

        for h in range(2):
            cp_o = pltpu.async_copy(obs_p.at[pairv.at[h]], obs_g, sem)
            cp_n = pltpu.async_copy(nobs_p.at[pairv.at[h]], nobs_g, sem)
            cp_o.wait()
            cp_n.wait()

            def assemble(b, _):
                bidx = jnp.full((L,), b, jnp.int32)
                iv = plsc.load_gather(idxv, [bidx + (h * HALF)])
                m = jnp.logical_and(iv == 0, any0)
                off = (iv & 1) * D_OBS
                rbase = (h * HALF + b) * D_OUT

                def seg(src_g, src_in, col0):
                    for c in range(D_OBS // L):
                        colv = iota16 + (c * L)
                        v = plsc.load_gather(src_g, [bidx, colv + off])
                        v = jnp.where(m, src_in[pl.ds(c * L, L)], v)
                        plsc.store_scatter(outv, [colv + (rbase + col0)], v)

                seg(obs_g, oin, 0)
                seg(nobs_g, nin, D_OBS + D_ACT + 1)

                av = plsc.load_gather(actv, [iv * D_ACT + iota16])
                av = jnp.where(m, ain[...], av)
                plsc.store_scatter(outv, [iota16 + (rbase + D_OBS)], av)
                return 0

            lax.fori_loop(0, HALF, assemble, 0)

        pltpu.sync_copy(outv, out_hbm.at[pl.ds(e * ENV_OUT, ENV_OUT)])
        return 0

    lax.fori_loop(0, ENV_PER_W, per_env, 0)


def kernel(observations, actions, rewards, next_observations,
           observations_in, actions_in, rewards_in, next_observations_in,
           indices, batch_size):
    del batch_size
    obs_p = observations.reshape(N_ENVS * BUF // 2, 2 * D_OBS)
    nobs_p = next_observations.reshape(N_ENVS * BUF // 2, 2 * D_OBS)
    act_f = actions.reshape(N_ENVS, BUF * D_ACT)

    mesh = plsc.VectorSubcoreMesh(core_axis_name="c", subcore_axis_name="s")
    run = pl.kernel(
        _body,
        out_type=jax.ShapeDtypeStruct((N_ENVS * BATCH * D_OUT,), jnp.float32),
        mesh=mesh,
        compiler_params=pltpu.CompilerParams(needs_layout_passes=False),
        scratch_types=[
            pltpu.VMEM((BATCH,), jnp.int32),
            pltpu.VMEM((2, HALF), jnp.int32),
            pltpu.VMEM((HALF, 2 * D_OBS), jnp.float32),
            pltpu.VMEM((HALF, 2 * D_OBS), jnp.float32),
            pltpu.VMEM((BUF * D_ACT,), jnp.float32),
            pltpu.VMEM((ENV_OUT,), jnp.float32),
            pltpu.VMEM((BUF,), jnp.float32),
            pltpu.VMEM((N_ENVS,), jnp.float32),
            pltpu.VMEM((D_OBS,), jnp.float32),
            pltpu.VMEM((D_ACT,), jnp.float32),
            pltpu.VMEM((D_OBS,), jnp.float32),
            pltpu.SemaphoreType.DMA,
        ],
    )
    out_flat = run(obs_p, act_f, rewards, nobs_p,
                   observations_in, actions_in, rewards_in, next_observations_in,
                   indices)
    return out_flat.reshape(N_ENVS * BATCH, D_OUT)


# speedup vs baseline: 1.3865x; 1.0888x over previous
import jax
import jax.numpy as jnp
from jax import lax
from jax.experimental import pallas as pl
from jax.experimental.pallas import tpu as pltpu
from jax.experimental.pallas import tpu_sc as plsc

N_ENVS = 1024
BUF = 1024
D_OBS = 64
D_ACT = 16
BATCH = 256
D_OUT = D_OBS + D_ACT + 1 + D_OBS
L = 16
N_WORKERS = 32
ENV_PER_W = N_ENVS // N_WORKERS
HALF = BATCH // 2
ENV_OUT = BATCH * D_OUT
HALF_OUT = HALF * D_OUT


def _body(obs_p, act_f, rew2, nobs_p, obs_in, act_in, rew_in1, nobs_in,
          idx_hbm, out_hbm,
          idxv, pairv, og0, ng0, og1, ng1, actv, outa, outb, rewrow, rinv,
          oin, ain, nin,
          sem_idx, sem_in, sem_act, sem_g0, sem_g1, sem_out):
    wid = lax.axis_index("s") * 2 + lax.axis_index("c")
    iota16 = lax.iota(jnp.int32, L)
    pltpu.sync_copy(rew_in1, rinv)
    halves = ((og0, ng0, outa, sem_g0), (og1, ng1, outb, sem_g1))

    def per_env(i, _):
        e = wid * ENV_PER_W + i
        ci = pltpu.async_copy(idx_hbm.at[e], idxv, sem_idx)
        c1 = pltpu.async_copy(obs_in.at[e], oin, sem_in)
        c2 = pltpu.async_copy(act_in.at[e], ain, sem_in)
        c3 = pltpu.async_copy(nobs_in.at[e], nin, sem_in)
        c4 = pltpu.async_copy(rew2.at[e], rewrow, sem_in)
        ca = pltpu.async_copy(act_f.at[e], actv, sem_act)

        ci.wait()
        ebase = (e * (BUF // 2)).astype(jnp.int32)
        nz = jnp.zeros((L,), jnp.int32)
        for c in range(BATCH // L):
            v = idxv[pl.ds(c * L, L)]
            h, r = divmod(c, HALF // L)
            pairv[h, pl.ds(r * L, L)] = ebase + lax.shift_right_logical(v, 1)
            nz = nz + jnp.where(v == 0, 1, 0).astype(jnp.int32)
        any0 = jnp.sum(nz) > 0

        gathers = []
        for h, (og, ng, _, sem_g) in enumerate(halves):
            gathers.append(pltpu.async_copy(obs_p.at[pairv.at[h]], og, sem_g))
            gathers.append(pltpu.async_copy(nobs_p.at[pairv.at[h]], ng, sem_g))
        c1.wait(); c2.wait(); c3.wait(); c4.wait(); ca.wait()
        rbc = plsc.load_gather(rinv, [jnp.full((L,), 1, jnp.int32) * e])

        for h, (og, ng, outh, sem_g) in enumerate(halves):
            @pl.when(i > 0)
            def _():
                pltpu.make_async_copy(
                    outh, out_hbm.at[pl.ds(0, HALF_OUT)], sem_out).wait()

            for c in range(HALF // L):
                ic = idxv[pl.ds(h * HALF + c * L, L)]
                g = plsc.load_gather(rewrow, [ic])
                g = jnp.where(ic == 0, rbc, g)
                plsc.store_scatter(
                    outh, [(iota16 + (c * L)) * D_OUT + (D_OBS + D_ACT)], g)

            gathers[2 * h].wait()
            gathers[2 * h + 1].wait()

            def make_asm(fix):
                def asm(b, _):
                    bidx = jnp.full((L,), b, jnp.int32)
                    iv = plsc.load_gather(idxv, [bidx + (h * HALF)])
                    off = (iv & 1) * D_OBS
                    rbase = b * D_OUT
                    m = iv == 0

                    def seg(src_g, src_in, col0):
                        for c in range(D_OBS // L):
                            colv = iota16 + (c * L)
                            v = plsc.load_gather(src_g, [bidx, colv + off])
                            if fix:
                                v = jnp.where(m, src_in[pl.ds(c * L, L)], v)
                            plsc.store_scatter(outh, [colv + (rbase + col0)], v)

                    seg(og, oin, 0)
                    seg(ng, nin, D_OBS + D_ACT + 1)
                    av = plsc.load_gather(actv, [iv * D_ACT + iota16])
                    if fix:
                        av = jnp.where(m, ain[...], av)
                    plsc.store_scatter(outh, [iota16 + (rbase + D_OBS)], av)
                    return 0
                return asm

            @pl.when(any0)
            def _():
                lax.fori_loop(0, HALF, make_asm(True), 0)

            @pl.when(jnp.logical_not(any0))
            def _():
                lax.fori_loop(0, HALF, make_asm(False), 0)

            pltpu.async_copy(
                outh, out_hbm.at[pl.ds(e * ENV_OUT + h * HALF_OUT, HALF_OUT)],
                sem_out)
        return 0

    lax.fori_loop(0, ENV_PER_W, per_env, 0)
    for outh in (outa, outb):
        pltpu.make_async_copy(outh, out_hbm.at[pl.ds(0, HALF_OUT)], sem_out).wait()


def kernel(observations, actions, rewards, next_observations,
           observations_in, actions_in, rewards_in, next_observations_in,
           indices, batch_size):
    del batch_size
    obs_p = observations.reshape(N_ENVS * BUF // 2, 2 * D_OBS)
    nobs_p = next_observations.reshape(N_ENVS * BUF // 2, 2 * D_OBS)
    act_f = actions.reshape(N_ENVS, BUF * D_ACT)

    mesh = plsc.VectorSubcoreMesh(core_axis_name="c", subcore_axis_name="s")
    run = pl.kernel(
        _body,
        out_type=jax.ShapeDtypeStruct((N_ENVS * BATCH * D_OUT,), jnp.float32),
        mesh=mesh,
        compiler_params=pltpu.CompilerParams(needs_layout_passes=False),
        scratch_types=[
            pltpu.VMEM((BATCH,), jnp.int32),
            pltpu.VMEM((2, HALF), jnp.int32),
            pltpu.VMEM((HALF, 2 * D_OBS), jnp.float32),
            pltpu.VMEM((HALF, 2 * D_OBS), jnp.float32),
            pltpu.VMEM((HALF, 2 * D_OBS), jnp.float32),
            pltpu.VMEM((HALF, 2 * D_OBS), jnp.float32),
            pltpu.VMEM((BUF * D_ACT,), jnp.float32),
            pltpu.VMEM((HALF_OUT,), jnp.float32),
            pltpu.VMEM((HALF_OUT,), jnp.float32),
            pltpu.VMEM((BUF,), jnp.float32),
            pltpu.VMEM((N_ENVS,), jnp.float32),
            pltpu.VMEM((D_OBS,), jnp.float32),
            pltpu.VMEM((D_ACT,), jnp.float32),
            pltpu.VMEM((D_OBS,), jnp.float32),
            pltpu.SemaphoreType.DMA,
            pltpu.SemaphoreType.DMA,
            pltpu.SemaphoreType.DMA,
            pltpu.SemaphoreType.DMA,
            pltpu.SemaphoreType.DMA,
            pltpu.SemaphoreType.DMA,
        ],
    )
    out_flat = run(obs_p, act_f, rewards, nobs_p,
                   observations_in, actions_in, rewards_in, next_observations_in,
                   indices)
    return out_flat.reshape(N_ENVS * BATCH, D_OUT)
